# baseline (speedup 1.0000x reference)
import functools

import jax
import jax.numpy as jnp
from jax import lax
from jax.experimental import pallas as pl
from jax.experimental.pallas import tpu as pltpu
from jax.experimental.pallas import tpu_sc as plsc
from jax.experimental.layout import Format, Layout

B, C, D, H, W = 2, 1, 128, 192, 192
HW = H * W
DHW = D * HW
N = B * DHW

ZP, YP, XP = D + 1, H + 1, W + 1
RY = XP
RZ = YP * XP
ROWS_B = ZP * RZ

NC, NS, LANES = 2, 16, 16
NW = NC * NS
NPW = N // NW
WPB = NW // B
K = 3072
ROWS_PER_CHUNK = K // W
GROUPS_PER_ROW = W // LANES
CHUNKS = NPW // K


def _floor_i32(v):
    ti = v.astype(jnp.int32)
    tf = ti.astype(jnp.float32)
    return jnp.where(tf > v, ti - 1, ti)


def _warp_body(table_hbm, flow_hbm, out_hbm, *rest):
    (dx0, dy0, dz0, fx0, fy0, fz0, ix0, vl0, ot0,
     dx1, dy1, dz1, fx1, fy1, fz1, ix1, vl1, ot1,
     semf, semg, semo) = rest
    S0 = (dx0, dy0, dz0, fx0, fy0, fz0, ix0, vl0, ot0)
    S1 = (dx1, dy1, dz1, fx1, fy1, fz1, ix1, vl1, ot1)

    cid = lax.axis_index("c")
    sid = lax.axis_index("s")
    wid = sid * NC + cid
    b = wid // WPB
    widx = wid % WPB
    o_batch0 = widx * NPW
    obase = b * DHW
    tbase = b * ROWS_B
    fbase = b * 3 * DHW

    def flow_srcs(g):
        o = fbase + o_batch0 + g * K
        return (flow_hbm.at[pl.ds(o, K)],
                flow_hbm.at[pl.ds(o + DHW, K)],
                flow_hbm.at[pl.ds(o + 2 * DHW, K)])

    def fire_flow(g, S):
        s0, s1, s2 = flow_srcs(g)
        pltpu.async_copy(s0, S[0], semf)
        pltpu.async_copy(s1, S[1], semf)
        pltpu.async_copy(s2, S[2], semf)

    def wait_flow(g, S):
        s0, s1, s2 = flow_srcs(g)
        pltpu.make_async_copy(s0, S[0], semf).wait()
        pltpu.make_async_copy(s1, S[1], semf).wait()
        pltpu.make_async_copy(s2, S[2], semf).wait()

    def fire_gather(S):
        pltpu.async_copy(table_hbm.at[S[6]], S[7], semg)

    def drain_gather(S):
        pltpu.make_async_copy(table_hbm.at[S[6]], S[7], semg).wait()

    def out_dst(g):
        return out_hbm.at[pl.ds(obase + o_batch0 + g * K, K)]

    def fire_out(g, S):
        pltpu.async_copy(S[8], out_dst(g), semo)

    def wait_out(g, S):
        pltpu.make_async_copy(S[8], out_dst(g), semo).wait()

    def pass1(g, S):
        dxv, dyv, dzv, fxv, fyv, fzv, ixv = S[0], S[1], S[2], S[3], S[4], S[5], S[6]
        o = o_batch0 + g * K
        row0 = o // W

        def row_body(t, c2):
            r = row0 + t
            z = r // H
            y = r - z * H
            yf = y.astype(jnp.float32)
            zf = z.astype(jnp.float32)
            for j in range(GROUPS_PER_ROW):
                sl = pl.ds(t * W + j * LANES, LANES)
                lane = lax.iota(jnp.int32, LANES).astype(jnp.float32) + float(j * LANES)
                xs = dxv[sl] + lane
                ys = dyv[sl] + yf
                zs = dzv[sl] + zf
                x0 = _floor_i32(xs)
                y0 = _floor_i32(ys)
                z0 = _floor_i32(zs)
                xp = jnp.clip(x0 + 1, 0, W)
                yp = jnp.clip(y0 + 1, 0, H)
                zp = jnp.clip(z0 + 1, 0, D)
                fxv[sl] = jnp.minimum(xp, W - 1).astype(jnp.float32) - xs
                fyv[sl] = jnp.minimum(yp, H - 1).astype(jnp.float32) - ys
                fzv[sl] = jnp.minimum(zp, D - 1).astype(jnp.float32) - zs
                ixv[sl] = zp * RZ + yp * RY + xp + tbase
            return c2

        lax.fori_loop(0, ROWS_PER_CHUNK, row_body, 0)

    iota16 = lax.iota(jnp.int32, LANES)
    cols = [jnp.full((LANES,), c, jnp.int32) for c in range(8)]

    def combine(S):
        fxv, fyv, fzv, vlv, otv = S[3], S[4], S[5], S[7], S[8]

        def comb_body(i, c2):
            sl = pl.ds(i * LANES, LANES)
            rows = i * LANES + iota16
            g = [plsc.load_gather(vlv, [rows, cols[c]]) for c in range(8)]
            fx = fxv[sl]
            fy = fyv[sl]
            fz = fzv[sl]
            ex = 1.0 - fx
            ey = 1.0 - fy
            ez = 1.0 - fz
            s00 = g[0] * fx + g[1] * ex
            s01 = g[2] * fx + g[3] * ex
            s10 = g[4] * fx + g[5] * ex
            s11 = g[6] * fx + g[7] * ex
            r0 = s00 * fy + s01 * ey
            r1 = s10 * fy + s11 * ey
            otv[sl] = r0 * fz + r1 * ez
            return c2

        lax.fori_loop(0, K // LANES, comb_body, 0)

    def steady(g, cur, prv):
        wait_flow(g, cur)
        pass1(g, cur)
        drain_gather(prv)
        fire_gather(cur)
        wait_out(g - 2, cur)
        combine(prv)
        fire_out(g - 1, prv)

        @pl.when(g + 1 < CHUNKS)
        def _():
            fire_flow(g + 1, prv)

    fire_flow(0, S0)
    wait_flow(0, S0)
    pass1(0, S0)
    fire_gather(S0)
    fire_flow(1, S1)
    wait_flow(1, S1)
    pass1(1, S1)
    drain_gather(S0)
    fire_gather(S1)
    combine(S0)
    fire_out(0, S0)
    fire_flow(2, S0)

    def pair_body(p, c2):
        steady(2 * p + 2, S0, S1)
        steady(2 * p + 3, S1, S0)
        return c2

    lax.fori_loop(0, CHUNKS // 2 - 1, pair_body, 0)

    drain_gather(S1)
    combine(S1)
    wait_out(CHUNKS - 2, S0)
    fire_out(CHUNKS - 1, S1)
    wait_out(CHUNKS - 1, S1)


@jax.jit
def _warp(I, flow):
    Ipad = jnp.pad(I[:, 0], ((0, 0), (1, 1), (1, 1), (1, 1)), mode="edge")
    parts = [Ipad[:, a:a + ZP, bb:bb + YP, cc:cc + XP]
             for a in (0, 1) for bb in (0, 1) for cc in (0, 1)]
    table = jnp.stack(parts, axis=-1).reshape(-1, 8)

    mesh = plsc.VectorSubcoreMesh(core_axis_name="c", subcore_axis_name="s")
    fbuf = [pltpu.VMEM((K,), jnp.float32)] * 6 + [
        pltpu.VMEM((K,), jnp.int32),
        pltpu.VMEM((K, 8), jnp.float32),
        pltpu.VMEM((K,), jnp.float32),
    ]
    f = functools.partial(
        pl.kernel,
        mesh=mesh,
        out_type=jax.ShapeDtypeStruct((N,), jnp.float32),
        scratch_types=fbuf + fbuf + [
            pltpu.SemaphoreType.DMA,
            pltpu.SemaphoreType.DMA,
            pltpu.SemaphoreType.DMA,
        ],
        compiler_params=pltpu.CompilerParams(
            needs_layout_passes=False, use_tc_tiling_on_sc=False),
    )(_warp_body)
    out = f(table, flow.reshape(-1))
    return out.reshape(B, C, D, H, W)


def kernel(I, flow):
    return _warp(I, flow)


# speedup vs baseline: 1.6701x; 1.6701x over previous
import functools

import jax
import jax.numpy as jnp
from jax import lax
from jax.experimental import pallas as pl
from jax.experimental.pallas import tpu as pltpu
from jax.experimental.pallas import tpu_sc as plsc
from jax.experimental.layout import Format, Layout

B, C, D, H, W = 2, 1, 128, 192, 192
HW = H * W
DHW = D * HW
N = B * DHW

ZP, YP, XP = D + 1, H + 1, W + 1
RY = XP
RZ = YP * XP
ROWS_B = ZP * RZ

NC, NS, LANES = 2, 16, 16
NW = NC * NS
NPW = N // NW
WPB = NW // B
K = 3072
ROWS_PER_CHUNK = K // W
GROUPS_PER_ROW = W // LANES
CHUNKS = NPW // K


def _floor_i32(v):
    ti = v.astype(jnp.int32)
    tf = ti.astype(jnp.float32)
    return jnp.where(tf > v, ti - 1, ti)


def _warp_body(table_hbm, flow_hbm, out_hbm, *rest):
    (dx0, dy0, dz0, fx0, fy0, fz0, ix0, vl0, ot0,
     dx1, dy1, dz1, fx1, fy1, fz1, ix1, vl1, ot1,
     semf, semg, semo) = rest
    S0 = (dx0, dy0, dz0, fx0, fy0, fz0, ix0, vl0, ot0)
    S1 = (dx1, dy1, dz1, fx1, fy1, fz1, ix1, vl1, ot1)

    cid = lax.axis_index("c")
    sid = lax.axis_index("s")
    wid = sid * NC + cid
    b = wid // WPB
    widx = wid % WPB
    o_batch0 = widx * NPW
    obase = b * DHW
    tbase = b * ROWS_B
    fbase = b * 3 * DHW

    def flow_srcs(g):
        o = fbase + o_batch0 + g * K
        return (flow_hbm.at[pl.ds(o, K)],
                flow_hbm.at[pl.ds(o + DHW, K)],
                flow_hbm.at[pl.ds(o + 2 * DHW, K)])

    def fire_flow(g, S):
        s0, s1, s2 = flow_srcs(g)
        pltpu.async_copy(s0, S[0], semf)
        pltpu.async_copy(s1, S[1], semf)
        pltpu.async_copy(s2, S[2], semf)

    def wait_flow(g, S):
        s0, s1, s2 = flow_srcs(g)
        pltpu.make_async_copy(s0, S[0], semf).wait()
        pltpu.make_async_copy(s1, S[1], semf).wait()
        pltpu.make_async_copy(s2, S[2], semf).wait()

    def fire_gather(S):
        pltpu.async_copy(table_hbm.at[S[6]], S[7], semg)

    def drain_gather(S):
        pltpu.make_async_copy(table_hbm.at[S[6]], S[7], semg).wait()

    def out_dst(g):
        return out_hbm.at[pl.ds(obase + o_batch0 + g * K, K)]

    def fire_out(g, S):
        pltpu.async_copy(S[8], out_dst(g), semo)

    def wait_out(g, S):
        pltpu.make_async_copy(S[8], out_dst(g), semo).wait()

    def pass1(g, S):
        dxv, dyv, dzv, fxv, fyv, fzv, ixv = S[0], S[1], S[2], S[3], S[4], S[5], S[6]
        o = o_batch0 + g * K
        row0 = o // W

        def row_body(t, c2):
            r = row0 + t
            z = r // H
            y = r - z * H
            yf = y.astype(jnp.float32)
            zf = z.astype(jnp.float32)
            for j in range(GROUPS_PER_ROW):
                sl = pl.ds(t * W + j * LANES, LANES)
                lane = lax.iota(jnp.int32, LANES).astype(jnp.float32) + float(j * LANES)
                xs = dxv[sl] + lane
                ys = dyv[sl] + yf
                zs = dzv[sl] + zf
                x0 = _floor_i32(xs)
                y0 = _floor_i32(ys)
                z0 = _floor_i32(zs)
                xp = jnp.clip(x0 + 1, 0, W)
                yp = jnp.clip(y0 + 1, 0, H)
                zp = jnp.clip(z0 + 1, 0, D)
                fxv[sl] = jnp.minimum(xp, W - 1).astype(jnp.float32) - xs
                fyv[sl] = jnp.minimum(yp, H - 1).astype(jnp.float32) - ys
                fzv[sl] = jnp.minimum(zp, D - 1).astype(jnp.float32) - zs
                ixv[sl] = zp * RZ + yp * RY + xp + tbase
            return c2

        lax.fori_loop(0, ROWS_PER_CHUNK, row_body, 0)

    iota16 = lax.iota(jnp.int32, LANES)
    cols = [jnp.full((LANES,), c, jnp.int32) for c in range(8)]

    def combine(S):
        fxv, fyv, fzv, vlv, otv = S[3], S[4], S[5], S[7], S[8]

        def comb_body(i, c2):
            sl = pl.ds(i * LANES, LANES)
            rows = i * LANES + iota16
            g = [plsc.load_gather(vlv, [rows, cols[c]]) for c in range(8)]
            fx = fxv[sl]
            fy = fyv[sl]
            fz = fzv[sl]
            ex = 1.0 - fx
            ey = 1.0 - fy
            ez = 1.0 - fz
            s00 = g[0] * fx + g[1] * ex
            s01 = g[2] * fx + g[3] * ex
            s10 = g[4] * fx + g[5] * ex
            s11 = g[6] * fx + g[7] * ex
            r0 = s00 * fy + s01 * ey
            r1 = s10 * fy + s11 * ey
            otv[sl] = r0 * fz + r1 * ez
            return c2

        lax.fori_loop(0, K // LANES, comb_body, 0)

    def steady(g, cur, prv):
        wait_flow(g, cur)
        pass1(g, cur)
        drain_gather(prv)
        fire_gather(cur)
        wait_out(g - 2, cur)
        combine(prv)
        fire_out(g - 1, prv)

        @pl.when(g + 1 < CHUNKS)
        def _():
            fire_flow(g + 1, prv)

    fire_flow(0, S0)
    wait_flow(0, S0)
    pass1(0, S0)
    fire_gather(S0)
    fire_flow(1, S1)
    wait_flow(1, S1)
    pass1(1, S1)
    drain_gather(S0)
    fire_gather(S1)
    combine(S0)
    fire_out(0, S0)
    fire_flow(2, S0)

    def pair_body(p, c2):
        steady(2 * p + 2, S0, S1)
        steady(2 * p + 3, S1, S0)
        return c2

    lax.fori_loop(0, CHUNKS // 2 - 1, pair_body, 0)

    drain_gather(S1)
    combine(S1)
    wait_out(CHUNKS - 2, S0)
    fire_out(CHUNKS - 1, S1)
    wait_out(CHUNKS - 1, S1)


@jax.jit
def _warp(I, flow):
    Ipad = jnp.pad(I[:, 0], ((0, 0), (1, 1), (1, 1), (1, 1)), mode="edge")
    parts = [Ipad[:, a:a + ZP, bb:bb + YP, cc:cc + XP]
             for a in (0, 1) for bb in (0, 1) for cc in (0, 1)]
    table = jnp.stack(parts, axis=-1).reshape(-1)
    table = lax.optimization_barrier(table).reshape(-1, 8)

    mesh = plsc.VectorSubcoreMesh(core_axis_name="c", subcore_axis_name="s")
    fbuf = [pltpu.VMEM((K,), jnp.float32)] * 6 + [
        pltpu.VMEM((K,), jnp.int32),
        pltpu.VMEM((K, 8), jnp.float32),
        pltpu.VMEM((K,), jnp.float32),
    ]
    f = functools.partial(
        pl.kernel,
        mesh=mesh,
        out_type=jax.ShapeDtypeStruct((N,), jnp.float32),
        scratch_types=fbuf + fbuf + [
            pltpu.SemaphoreType.DMA,
            pltpu.SemaphoreType.DMA,
            pltpu.SemaphoreType.DMA,
        ],
        compiler_params=pltpu.CompilerParams(
            needs_layout_passes=False, use_tc_tiling_on_sc=False),
    )(_warp_body)
    out = f(table, flow.reshape(-1))
    return out.reshape(B, C, D, H, W)


def kernel(I, flow):
    return _warp(I, flow)


# speedup vs baseline: 6.3675x; 3.8126x over previous
import functools

import jax
import jax.numpy as jnp
from jax import lax
from jax.experimental import pallas as pl
from jax.experimental.pallas import tpu as pltpu
from jax.experimental.pallas import tpu_sc as plsc
from jax.experimental.layout import Format, Layout

B, C, D, H, W = 2, 1, 128, 192, 192
HW = H * W
DHW = D * HW
N = B * DHW

ZP, YP, XP = D + 1, H + 1, W + 1
RY = XP
RZ = YP * XP
ROWS_B = ZP * RZ

NC, NS, LANES = 2, 16, 16
NW = NC * NS
NPW = N // NW
WPB = NW // B
K = 3072
ROWS_PER_CHUNK = K // W
GROUPS_PER_ROW = W // LANES
CHUNKS = NPW // K


NR = 25
NY = NR + 1
NPLANES = B * ZP
NCH = (YP + NR - 1) // NR
NITEMS = NPLANES * NCH
TAIL = YP - (NCH - 1) * NR


def _build_body(I_hbm, table_hbm, src0, src1, outbuf):
    cid = lax.axis_index("c")
    sid = lax.axis_index("s")
    wid = sid * NC + cid

    iota16 = lax.iota(jnp.int32, LANES)

    def item_body(it, c1):
        w = wid + it * NW

        @pl.when(w < NITEMS)
        def _():
            plane = w // NCH
            ch = w - plane * NCH
            b = plane // ZP
            zp = plane - b * ZP
            yp0 = ch * NR
            full = ch < NCH - 1
            z0s = jnp.clip(zp - 1, 0, D - 1)
            z1s = jnp.clip(zp, 0, D - 1)
            ys0 = jnp.clip(yp0 - 1, 0, H - NY)
            ibase = b * DHW + ys0 * W
            pltpu.sync_copy(I_hbm.at[pl.ds(ibase + z0s * HW, NY * W)], src0)
            pltpu.sync_copy(I_hbm.at[pl.ds(ibase + z1s * HW, NY * W)], src1)

            def row_body(t, c2):
                yp = yp0 + t
                b0 = (jnp.clip(yp - 1, 0, H - 1) - ys0) * W
                b1 = (jnp.clip(yp, 0, H - 1) - ys0) * W
                for g in range(13):
                    xpb = g * 16 if g < 12 else XP - 16
                    xv = xpb + iota16
                    xi0 = jnp.clip(xv - 1, 0, W - 1)
                    xi1 = jnp.clip(xv, 0, W - 1)
                    rows = t * XP + xv
                    for c, (src, yb, xi) in enumerate(
                        (src, yb, xi)
                        for src in (src0, src1)
                        for yb in (b0, b1)
                        for xi in (xi0, xi1)
                    ):
                        v = plsc.load_gather(src, [yb + xi])
                        plsc.store_scatter(
                            outbuf, [rows, jnp.full((LANES,), c, jnp.int32)], v)
                return c2

            nrows = jnp.where(full, NR, TAIL)
            lax.fori_loop(0, nrows, row_body, 0)
            dst0 = ((b * ZP + zp) * YP + yp0) * XP

            @pl.when(full)
            def _():
                pltpu.sync_copy(outbuf.at[pl.ds(0, NR * XP), :],
                                table_hbm.at[pl.ds(dst0, NR * XP), :])

            @pl.when(jnp.logical_not(full))
            def _():
                pltpu.sync_copy(outbuf.at[pl.ds(0, TAIL * XP), :],
                                table_hbm.at[pl.ds(dst0, TAIL * XP), :])

        return c1

    lax.fori_loop(0, (NITEMS + NW - 1) // NW, item_body, 0)


def _floor_i32(v):
    ti = v.astype(jnp.int32)
    tf = ti.astype(jnp.float32)
    return jnp.where(tf > v, ti - 1, ti)


def _warp_body(table_hbm, flow_hbm, out_hbm, *rest):
    (dx0, dy0, dz0, fx0, fy0, fz0, ix0, vl0, ot0,
     dx1, dy1, dz1, fx1, fy1, fz1, ix1, vl1, ot1,
     semf, semg, semo) = rest
    S0 = (dx0, dy0, dz0, fx0, fy0, fz0, ix0, vl0, ot0)
    S1 = (dx1, dy1, dz1, fx1, fy1, fz1, ix1, vl1, ot1)

    cid = lax.axis_index("c")
    sid = lax.axis_index("s")
    wid = sid * NC + cid
    b = wid // WPB
    widx = wid % WPB
    o_batch0 = widx * NPW
    obase = b * DHW
    tbase = b * ROWS_B
    fbase = b * 3 * DHW

    def flow_srcs(g):
        o = fbase + o_batch0 + g * K
        return (flow_hbm.at[pl.ds(o, K)],
                flow_hbm.at[pl.ds(o + DHW, K)],
                flow_hbm.at[pl.ds(o + 2 * DHW, K)])

    def fire_flow(g, S):
        s0, s1, s2 = flow_srcs(g)
        pltpu.async_copy(s0, S[0], semf)
        pltpu.async_copy(s1, S[1], semf)
        pltpu.async_copy(s2, S[2], semf)

    def wait_flow(g, S):
        s0, s1, s2 = flow_srcs(g)
        pltpu.make_async_copy(s0, S[0], semf).wait()
        pltpu.make_async_copy(s1, S[1], semf).wait()
        pltpu.make_async_copy(s2, S[2], semf).wait()

    def fire_gather(S):
        pltpu.async_copy(table_hbm.at[S[6]], S[7], semg)

    def drain_gather(S):
        pltpu.make_async_copy(table_hbm.at[S[6]], S[7], semg).wait()

    def out_dst(g):
        return out_hbm.at[pl.ds(obase + o_batch0 + g * K, K)]

    def fire_out(g, S):
        pltpu.async_copy(S[8], out_dst(g), semo)

    def wait_out(g, S):
        pltpu.make_async_copy(S[8], out_dst(g), semo).wait()

    def pass1(g, S):
        dxv, dyv, dzv, fxv, fyv, fzv, ixv = S[0], S[1], S[2], S[3], S[4], S[5], S[6]
        o = o_batch0 + g * K
        row0 = o // W

        def row_body(t, c2):
            r = row0 + t
            z = r // H
            y = r - z * H
            yf = y.astype(jnp.float32)
            zf = z.astype(jnp.float32)
            for j in range(GROUPS_PER_ROW):
                sl = pl.ds(t * W + j * LANES, LANES)
                lane = lax.iota(jnp.int32, LANES).astype(jnp.float32) + float(j * LANES)
                xs = dxv[sl] + lane
                ys = dyv[sl] + yf
                zs = dzv[sl] + zf
                x0 = _floor_i32(xs)
                y0 = _floor_i32(ys)
                z0 = _floor_i32(zs)
                xp = jnp.clip(x0 + 1, 0, W)
                yp = jnp.clip(y0 + 1, 0, H)
                zp = jnp.clip(z0 + 1, 0, D)
                fxv[sl] = jnp.minimum(xp, W - 1).astype(jnp.float32) - xs
                fyv[sl] = jnp.minimum(yp, H - 1).astype(jnp.float32) - ys
                fzv[sl] = jnp.minimum(zp, D - 1).astype(jnp.float32) - zs
                ixv[sl] = zp * RZ + yp * RY + xp + tbase
            return c2

        lax.fori_loop(0, ROWS_PER_CHUNK, row_body, 0)

    iota16 = lax.iota(jnp.int32, LANES)
    cols = [jnp.full((LANES,), c, jnp.int32) for c in range(8)]

    def combine(S):
        fxv, fyv, fzv, vlv, otv = S[3], S[4], S[5], S[7], S[8]

        def comb_body(i, c2):
            sl = pl.ds(i * LANES, LANES)
            rows = i * LANES + iota16
            g = [plsc.load_gather(vlv, [rows, cols[c]]) for c in range(8)]
            fx = fxv[sl]
            fy = fyv[sl]
            fz = fzv[sl]
            ex = 1.0 - fx
            ey = 1.0 - fy
            ez = 1.0 - fz
            s00 = g[0] * fx + g[1] * ex
            s01 = g[2] * fx + g[3] * ex
            s10 = g[4] * fx + g[5] * ex
            s11 = g[6] * fx + g[7] * ex
            r0 = s00 * fy + s01 * ey
            r1 = s10 * fy + s11 * ey
            otv[sl] = r0 * fz + r1 * ez
            return c2

        lax.fori_loop(0, K // LANES, comb_body, 0)

    def steady(g, cur, prv):
        wait_flow(g, cur)
        pass1(g, cur)
        drain_gather(prv)
        fire_gather(cur)
        wait_out(g - 2, cur)
        combine(prv)
        fire_out(g - 1, prv)

        @pl.when(g + 1 < CHUNKS)
        def _():
            fire_flow(g + 1, prv)

    fire_flow(0, S0)
    wait_flow(0, S0)
    pass1(0, S0)
    fire_gather(S0)
    fire_flow(1, S1)
    wait_flow(1, S1)
    pass1(1, S1)
    drain_gather(S0)
    fire_gather(S1)
    combine(S0)
    fire_out(0, S0)
    fire_flow(2, S0)

    def pair_body(p, c2):
        steady(2 * p + 2, S0, S1)
        steady(2 * p + 3, S1, S0)
        return c2

    lax.fori_loop(0, CHUNKS // 2 - 1, pair_body, 0)

    drain_gather(S1)
    combine(S1)
    wait_out(CHUNKS - 2, S0)
    fire_out(CHUNKS - 1, S1)
    wait_out(CHUNKS - 1, S1)


@jax.jit
def _warp(I, flow):
    mesh = plsc.VectorSubcoreMesh(core_axis_name="c", subcore_axis_name="s")
    build = functools.partial(
        pl.kernel,
        mesh=mesh,
        out_type=jax.ShapeDtypeStruct((B * ROWS_B, 8), jnp.float32),
        scratch_types=[
            pltpu.VMEM((NY * W,), jnp.float32),
            pltpu.VMEM((NY * W,), jnp.float32),
            pltpu.VMEM((NR * XP, 8), jnp.float32),
        ],
        compiler_params=pltpu.CompilerParams(
            needs_layout_passes=False, use_tc_tiling_on_sc=False),
    )(_build_body)
    table = build(I.reshape(-1))
    fbuf = [pltpu.VMEM((K,), jnp.float32)] * 6 + [
        pltpu.VMEM((K,), jnp.int32),
        pltpu.VMEM((K, 8), jnp.float32),
        pltpu.VMEM((K,), jnp.float32),
    ]
    f = functools.partial(
        pl.kernel,
        mesh=mesh,
        out_type=jax.ShapeDtypeStruct((N,), jnp.float32),
        scratch_types=fbuf + fbuf + [
            pltpu.SemaphoreType.DMA,
            pltpu.SemaphoreType.DMA,
            pltpu.SemaphoreType.DMA,
        ],
        compiler_params=pltpu.CompilerParams(
            needs_layout_passes=False, use_tc_tiling_on_sc=False),
    )(_warp_body)
    out = f(table, flow.reshape(-1))
    return out.reshape(B, C, D, H, W)


def kernel(I, flow):
    return _warp(I, flow)


# speedup vs baseline: 6.4888x; 1.0190x over previous
import functools

import jax
import jax.numpy as jnp
from jax import lax
from jax.experimental import pallas as pl
from jax.experimental.pallas import tpu as pltpu
from jax.experimental.pallas import tpu_sc as plsc

B, C, D, H, W = 2, 1, 128, 192, 192
HW = H * W
DHW = D * HW
N = B * DHW

ZP, YP, XP = D + 1, H + 1, W + 1
RY = XP
RZ = YP * XP
ROWS_B = ZP * RZ

NC, NS, LANES = 2, 16, 16
NW = NC * NS
NPW = N // NW
WPB = NW // B
K = 1536
ROWS_PER_CHUNK = K // W
GROUPS_PER_ROW = W // LANES
CHUNKS = NPW // K

NR = 49
NY = NR + 1
NPLANES = B * ZP
NCH = (YP + NR - 1) // NR
NITEMS = NPLANES * NCH
TAIL = YP - (NCH - 1) * NR


def _build_body(I_hbm, table_hbm, src0, src1, outbuf):
    cid = lax.axis_index("c")
    sid = lax.axis_index("s")
    wid = sid * NC + cid

    iota16 = lax.iota(jnp.int32, LANES)

    def item_body(it, c1):
        w = wid + it * NW

        @pl.when(w < NITEMS)
        def _():
            plane = w // NCH
            ch = w - plane * NCH
            b = plane // ZP
            zp = plane - b * ZP
            yp0 = ch * NR
            full = ch < NCH - 1
            z0s = jnp.clip(zp - 1, 0, D - 1)
            z1s = jnp.clip(zp, 0, D - 1)
            ys0 = jnp.clip(yp0 - 1, 0, H - NY)
            ibase = b * DHW + ys0 * W
            pltpu.sync_copy(I_hbm.at[pl.ds(ibase + z0s * HW, NY * W)], src0)
            pltpu.sync_copy(I_hbm.at[pl.ds(ibase + z1s * HW, NY * W)], src1)

            def row_body(t, c2):
                yp = yp0 + t
                b0 = (jnp.clip(yp - 1, 0, H - 1) - ys0) * W
                b1 = (jnp.clip(yp, 0, H - 1) - ys0) * W
                for g in range(13):
                    xpb = g * 16 if g < 12 else XP - 16
                    xv = xpb + iota16
                    xi0 = jnp.clip(xv - 1, 0, W - 1)
                    xi1 = jnp.clip(xv, 0, W - 1)
                    rows = t * XP + xv
                    combos = [(src, yb, xi)
                              for src in (src0, src1)
                              for yb in (b0, b1)
                              for xi in (xi0, xi1)]
                    for c, (src, yb, xi) in enumerate(combos):
                        v = plsc.load_gather(src, [yb + xi])
                        plsc.store_scatter(
                            outbuf, [rows, jnp.full((LANES,), c, jnp.int32)], v)
                return c2

            nrows = jnp.where(full, NR, TAIL)
            lax.fori_loop(0, nrows, row_body, 0)
            dst0 = ((b * ZP + zp) * YP + yp0) * XP

            @pl.when(full)
            def _():
                pltpu.sync_copy(outbuf.at[pl.ds(0, NR * XP), :],
                                table_hbm.at[pl.ds(dst0, NR * XP), :])

            @pl.when(jnp.logical_not(full))
            def _():
                pltpu.sync_copy(outbuf.at[pl.ds(0, TAIL * XP), :],
                                table_hbm.at[pl.ds(dst0, TAIL * XP), :])

        return c1

    lax.fori_loop(0, (NITEMS + NW - 1) // NW, item_body, 0)


def _floor_i32(v):
    ti = v.astype(jnp.int32)
    tf = ti.astype(jnp.float32)
    return jnp.where(tf > v, ti - 1, ti)


def _warp_body(table_hbm, flow_hbm, out_hbm, *rest):
    sets = [rest[i * 9:(i + 1) * 9] for i in range(3)]
    semf, semg, semo = rest[27:30]

    cid = lax.axis_index("c")
    sid = lax.axis_index("s")
    wid = sid * NC + cid
    b = wid // WPB
    widx = wid % WPB
    o_batch0 = widx * NPW
    obase = b * DHW
    tbase = b * ROWS_B
    fbase = b * 3 * DHW

    def flow_srcs(g):
        o = fbase + o_batch0 + g * K
        return (flow_hbm.at[pl.ds(o, K)],
                flow_hbm.at[pl.ds(o + DHW, K)],
                flow_hbm.at[pl.ds(o + 2 * DHW, K)])

    def fire_flow(g, S):
        s0, s1, s2 = flow_srcs(g)
        pltpu.async_copy(s0, S[0], semf)
        pltpu.async_copy(s1, S[1], semf)
        pltpu.async_copy(s2, S[2], semf)

    def wait_flow(g, S):
        s0, s1, s2 = flow_srcs(g)
        pltpu.make_async_copy(s0, S[0], semf).wait()
        pltpu.make_async_copy(s1, S[1], semf).wait()
        pltpu.make_async_copy(s2, S[2], semf).wait()

    def fire_gather(S):
        pltpu.async_copy(table_hbm.at[S[6]], S[7], semg)

    def drain_gather(S):
        pltpu.make_async_copy(table_hbm.at[S[6]], S[7], semg).wait()

    def out_dst(g):
        return out_hbm.at[pl.ds(obase + o_batch0 + g * K, K)]

    def fire_out(g, S):
        pltpu.async_copy(S[8], out_dst(g), semo)

    def wait_out(g, S):
        pltpu.make_async_copy(S[8], out_dst(g), semo).wait()

    def pass1(g, S):
        dxv, dyv, dzv, fxv, fyv, fzv, ixv = S[0], S[1], S[2], S[3], S[4], S[5], S[6]
        o = o_batch0 + g * K
        row0 = o // W

        def row_body(t, c2):
            r = row0 + t
            z = r // H
            y = r - z * H
            yf = y.astype(jnp.float32)
            zf = z.astype(jnp.float32)
            for j in range(GROUPS_PER_ROW):
                sl = pl.ds(t * W + j * LANES, LANES)
                lane = lax.iota(jnp.int32, LANES).astype(jnp.float32) + float(j * LANES)
                xs = dxv[sl] + lane
                ys = dyv[sl] + yf
                zs = dzv[sl] + zf
                x0 = _floor_i32(xs)
                y0 = _floor_i32(ys)
                z0 = _floor_i32(zs)
                xp = jnp.clip(x0 + 1, 0, W)
                yp = jnp.clip(y0 + 1, 0, H)
                zp = jnp.clip(z0 + 1, 0, D)
                fxv[sl] = jnp.minimum(xp, W - 1).astype(jnp.float32) - xs
                fyv[sl] = jnp.minimum(yp, H - 1).astype(jnp.float32) - ys
                fzv[sl] = jnp.minimum(zp, D - 1).astype(jnp.float32) - zs
                ixv[sl] = zp * RZ + yp * RY + xp + tbase
            return c2

        lax.fori_loop(0, ROWS_PER_CHUNK, row_body, 0)

    iota16 = lax.iota(jnp.int32, LANES)
    cols = [jnp.full((LANES,), c, jnp.int32) for c in range(8)]

    def combine(S):
        fxv, fyv, fzv, vlv, otv = S[3], S[4], S[5], S[7], S[8]

        def comb_body(i, c2):
            sl = pl.ds(i * LANES, LANES)
            rows = i * LANES + iota16
            g = [plsc.load_gather(vlv, [rows, cols[c]]) for c in range(8)]
            fx = fxv[sl]
            fy = fyv[sl]
            fz = fzv[sl]
            ex = 1.0 - fx
            ey = 1.0 - fy
            ez = 1.0 - fz
            s00 = g[0] * fx + g[1] * ex
            s01 = g[2] * fx + g[3] * ex
            s10 = g[4] * fx + g[5] * ex
            s11 = g[6] * fx + g[7] * ex
            r0 = s00 * fy + s01 * ey
            r1 = s10 * fy + s11 * ey
            otv[sl] = r0 * fz + r1 * ez
            return c2

        lax.fori_loop(0, K // LANES, comb_body, 0)

    def steady(g, cur, prv2, nxt):
        @pl.when(g < CHUNKS)
        def _():
            wait_flow(g, cur)
            pass1(g, cur)
            fire_gather(cur)

        @pl.when(g >= 2)
        def _():
            drain_gather(prv2)

            @pl.when(g >= 5)
            def _():
                wait_out(g - 5, prv2)

            combine(prv2)
            fire_out(g - 2, prv2)

        @pl.when(g + 1 < CHUNKS)
        def _():
            fire_flow(g + 1, nxt)

    S0, S1, S2 = sets
    fire_flow(0, S0)

    def triple_body(p, c1):
        g = 3 * p
        steady(g, S0, S1, S1)
        steady(g + 1, S1, S2, S2)
        steady(g + 2, S2, S0, S0)
        return c1

    lax.fori_loop(0, CHUNKS // 3, triple_body, 0)

    steady(jnp.int32(CHUNKS), S0, S1, S1)
    steady(jnp.int32(CHUNKS + 1), S1, S2, S2)
    wait_out(CHUNKS - 3, S0)
    wait_out(CHUNKS - 2, S1)
    wait_out(CHUNKS - 1, S2)


@jax.jit
def _warp(I, flow):
    mesh = plsc.VectorSubcoreMesh(core_axis_name="c", subcore_axis_name="s")
    params = pltpu.CompilerParams(
        needs_layout_passes=False, use_tc_tiling_on_sc=False)
    build = functools.partial(
        pl.kernel,
        mesh=mesh,
        out_type=jax.ShapeDtypeStruct((B * ROWS_B, 8), jnp.float32),
        scratch_types=[
            pltpu.VMEM((NY * W,), jnp.float32),
            pltpu.VMEM((NY * W,), jnp.float32),
            pltpu.VMEM((NR * XP, 8), jnp.float32),
        ],
        compiler_params=params,
    )(_build_body)
    table = build(I.reshape(-1))

    sbuf = [
        pltpu.VMEM((K,), jnp.float32),
        pltpu.VMEM((K,), jnp.float32),
        pltpu.VMEM((K,), jnp.float32),
        pltpu.VMEM((K,), jnp.float32),
        pltpu.VMEM((K,), jnp.float32),
        pltpu.VMEM((K,), jnp.float32),
        pltpu.VMEM((K,), jnp.int32),
        pltpu.VMEM((K, 8), jnp.float32),
        pltpu.VMEM((K,), jnp.float32),
    ]
    f = functools.partial(
        pl.kernel,
        mesh=mesh,
        out_type=jax.ShapeDtypeStruct((N,), jnp.float32),
        scratch_types=sbuf * 3 + [
            pltpu.SemaphoreType.DMA,
            pltpu.SemaphoreType.DMA,
            pltpu.SemaphoreType.DMA,
        ],
        compiler_params=params,
    )(_warp_body)
    out = f(table, flow.reshape(-1))
    return out.reshape(B, C, D, H, W)


def kernel(I, flow):
    return _warp(I, flow)


# speedup vs baseline: 9.3822x; 1.4459x over previous
import functools

import jax
import jax.numpy as jnp
from jax import lax
from jax.experimental import pallas as pl
from jax.experimental.pallas import tpu as pltpu
from jax.experimental.pallas import tpu_sc as plsc

B, C, D, H, W = 2, 1, 128, 192, 192
HW = H * W
DHW = D * HW
N = B * DHW

ZP, YP, XP = D + 1, H + 1, W + 1
RY = XP
RZ = YP * XP
ROWS_B = ZP * RZ

NC, NS, LANES = 2, 16, 16
NW = NC * NS
NPW = N // NW
WPB = NW // B
K = 1536
ROWS_PER_CHUNK = K // W
GROUPS_PER_ROW = W // LANES
CHUNKS = NPW // K

NR = 49
NY = NR + 1
NPLANES = B * ZP
NCH = (YP + NR - 1) // NR
NITEMS = NPLANES * NCH
TAIL = YP - (NCH - 1) * NR


def _build_body(I_hbm, table_hbm, src0, src1, outbuf):
    cid = lax.axis_index("c")
    sid = lax.axis_index("s")
    wid = sid * NC + cid

    iota16 = lax.iota(jnp.int32, LANES)

    def item_body(it, c1):
        w = wid + it * NW

        @pl.when(w < NITEMS)
        def _():
            plane = w // NCH
            ch = w - plane * NCH
            b = plane // ZP
            zp = plane - b * ZP
            yp0 = ch * NR
            full = ch < NCH - 1
            z0s = jnp.clip(zp - 1, 0, D - 1)
            z1s = jnp.clip(zp, 0, D - 1)
            ys0 = jnp.clip(yp0 - 1, 0, H - NY)
            ibase = b * DHW + ys0 * W
            pltpu.sync_copy(I_hbm.at[pl.ds(ibase + z0s * HW, NY * W)], src0)
            pltpu.sync_copy(I_hbm.at[pl.ds(ibase + z1s * HW, NY * W)], src1)

            def row_body(t):
                yp = yp0 + t
                b0 = (jnp.clip(yp - 1, 0, H - 1) - ys0) * W
                b1 = (jnp.clip(yp, 0, H - 1) - ys0) * W
                for g in range(13):
                    xpb = g * 16 if g < 12 else XP - 16
                    xv = xpb + iota16
                    xi0 = jnp.clip(xv - 1, 0, W - 1)
                    xi1 = jnp.clip(xv, 0, W - 1)
                    rows = t * XP + xv
                    combos = [(src, yb, xi)
                              for src in (src0, src1)
                              for yb in (b0, b1)
                              for xi in (xi0, xi1)]
                    vs = [plsc.load_gather(src, [yb + xi])
                          for (src, yb, xi) in combos]
                    for c, v in enumerate(vs):
                        plsc.store_scatter(
                            outbuf, [rows, jnp.full((LANES,), c, jnp.int32)], v)

            nrows = jnp.where(full, NR, TAIL)
            plsc.parallel_loop(0, nrows)(row_body)
            dst0 = ((b * ZP + zp) * YP + yp0) * XP

            @pl.when(full)
            def _():
                pltpu.sync_copy(outbuf.at[pl.ds(0, NR * XP), :],
                                table_hbm.at[pl.ds(dst0, NR * XP), :])

            @pl.when(jnp.logical_not(full))
            def _():
                pltpu.sync_copy(outbuf.at[pl.ds(0, TAIL * XP), :],
                                table_hbm.at[pl.ds(dst0, TAIL * XP), :])

        return c1

    lax.fori_loop(0, (NITEMS + NW - 1) // NW, item_body, 0)


def _floor_i32(v):
    ti = v.astype(jnp.int32)
    tf = ti.astype(jnp.float32)
    return jnp.where(tf > v, ti - 1, ti)


def _warp_body(table_hbm, flow_hbm, out_hbm, *rest):
    sets = [rest[i * 9:(i + 1) * 9] for i in range(3)]
    semf, semg, semo = rest[27:30]

    cid = lax.axis_index("c")
    sid = lax.axis_index("s")
    wid = sid * NC + cid
    b = wid // WPB
    widx = wid % WPB
    o_batch0 = widx * NPW
    obase = b * DHW
    tbase = b * ROWS_B
    fbase = b * 3 * DHW

    def flow_srcs(g):
        o = fbase + o_batch0 + g * K
        return (flow_hbm.at[pl.ds(o, K)],
                flow_hbm.at[pl.ds(o + DHW, K)],
                flow_hbm.at[pl.ds(o + 2 * DHW, K)])

    def fire_flow(g, S):
        s0, s1, s2 = flow_srcs(g)
        pltpu.async_copy(s0, S[0], semf)
        pltpu.async_copy(s1, S[1], semf)
        pltpu.async_copy(s2, S[2], semf)

    def wait_flow(g, S):
        s0, s1, s2 = flow_srcs(g)
        pltpu.make_async_copy(s0, S[0], semf).wait()
        pltpu.make_async_copy(s1, S[1], semf).wait()
        pltpu.make_async_copy(s2, S[2], semf).wait()

    def fire_gather(S):
        pltpu.async_copy(table_hbm.at[S[6]], S[7], semg)

    def drain_gather(S):
        pltpu.make_async_copy(table_hbm.at[S[6]], S[7], semg).wait()

    def out_dst(g):
        return out_hbm.at[pl.ds(obase + o_batch0 + g * K, K)]

    def fire_out(g, S):
        pltpu.async_copy(S[8], out_dst(g), semo)

    def wait_out(g, S):
        pltpu.make_async_copy(S[8], out_dst(g), semo).wait()

    def pass1(g, S):
        dxv, dyv, dzv, fxv, fyv, fzv, ixv = S[0], S[1], S[2], S[3], S[4], S[5], S[6]
        o = o_batch0 + g * K
        row0 = o // W

        def row_body(t):
            r = row0 + t
            z = r // H
            y = r - z * H
            yf = y.astype(jnp.float32)
            zf = z.astype(jnp.float32)
            for j in range(GROUPS_PER_ROW):
                sl = pl.ds(t * W + j * LANES, LANES)
                lane = lax.iota(jnp.int32, LANES).astype(jnp.float32) + float(j * LANES)
                xs = dxv[sl] + lane
                ys = dyv[sl] + yf
                zs = dzv[sl] + zf
                x0 = _floor_i32(xs)
                y0 = _floor_i32(ys)
                z0 = _floor_i32(zs)
                xp = jnp.clip(x0 + 1, 0, W)
                yp = jnp.clip(y0 + 1, 0, H)
                zp = jnp.clip(z0 + 1, 0, D)
                fxv[sl] = jnp.minimum(xp, W - 1).astype(jnp.float32) - xs
                fyv[sl] = jnp.minimum(yp, H - 1).astype(jnp.float32) - ys
                fzv[sl] = jnp.minimum(zp, D - 1).astype(jnp.float32) - zs
                ixv[sl] = zp * RZ + yp * RY + xp + tbase

        plsc.parallel_loop(0, ROWS_PER_CHUNK)(row_body)

    iota16 = lax.iota(jnp.int32, LANES)
    cols = [jnp.full((LANES,), c, jnp.int32) for c in range(8)]

    def combine(S):
        fxv, fyv, fzv, vlv, otv = S[3], S[4], S[5], S[7], S[8]

        def comb_body(i):
            sl = pl.ds(i * LANES, LANES)
            rows = i * LANES + iota16
            g = [plsc.load_gather(vlv, [rows, cols[c]]) for c in range(8)]
            fx = fxv[sl]
            fy = fyv[sl]
            fz = fzv[sl]
            ex = 1.0 - fx
            ey = 1.0 - fy
            ez = 1.0 - fz
            s00 = g[0] * fx + g[1] * ex
            s01 = g[2] * fx + g[3] * ex
            s10 = g[4] * fx + g[5] * ex
            s11 = g[6] * fx + g[7] * ex
            r0 = s00 * fy + s01 * ey
            r1 = s10 * fy + s11 * ey
            otv[sl] = r0 * fz + r1 * ez

        plsc.parallel_loop(0, K // LANES)(comb_body)

    def steady(g, cur, prv2, nxt):
        @pl.when(g < CHUNKS)
        def _():
            wait_flow(g, cur)
            pass1(g, cur)
            fire_gather(cur)

        @pl.when(g >= 2)
        def _():
            drain_gather(prv2)

            @pl.when(g >= 5)
            def _():
                wait_out(g - 5, prv2)

            combine(prv2)
            fire_out(g - 2, prv2)

        @pl.when(g + 1 < CHUNKS)
        def _():
            fire_flow(g + 1, nxt)

    S0, S1, S2 = sets
    fire_flow(0, S0)

    def triple_body(p, c1):
        g = 3 * p
        steady(g, S0, S1, S1)
        steady(g + 1, S1, S2, S2)
        steady(g + 2, S2, S0, S0)
        return c1

    lax.fori_loop(0, CHUNKS // 3, triple_body, 0)

    steady(jnp.int32(CHUNKS), S0, S1, S1)
    steady(jnp.int32(CHUNKS + 1), S1, S2, S2)
    wait_out(CHUNKS - 3, S0)
    wait_out(CHUNKS - 2, S1)
    wait_out(CHUNKS - 1, S2)


@jax.jit
def _warp(I, flow):
    mesh = plsc.VectorSubcoreMesh(core_axis_name="c", subcore_axis_name="s")
    params = pltpu.CompilerParams(
        needs_layout_passes=False, use_tc_tiling_on_sc=False)
    build = functools.partial(
        pl.kernel,
        mesh=mesh,
        out_type=jax.ShapeDtypeStruct((B * ROWS_B, 8), jnp.float32),
        scratch_types=[
            pltpu.VMEM((NY * W,), jnp.float32),
            pltpu.VMEM((NY * W,), jnp.float32),
            pltpu.VMEM((NR * XP, 8), jnp.float32),
        ],
        compiler_params=params,
    )(_build_body)
    table = build(I.reshape(-1))

    sbuf = [
        pltpu.VMEM((K,), jnp.float32),
        pltpu.VMEM((K,), jnp.float32),
        pltpu.VMEM((K,), jnp.float32),
        pltpu.VMEM((K,), jnp.float32),
        pltpu.VMEM((K,), jnp.float32),
        pltpu.VMEM((K,), jnp.float32),
        pltpu.VMEM((K,), jnp.int32),
        pltpu.VMEM((K, 8), jnp.float32),
        pltpu.VMEM((K,), jnp.float32),
    ]
    f = functools.partial(
        pl.kernel,
        mesh=mesh,
        out_type=jax.ShapeDtypeStruct((N,), jnp.float32),
        scratch_types=sbuf * 3 + [
            pltpu.SemaphoreType.DMA,
            pltpu.SemaphoreType.DMA,
            pltpu.SemaphoreType.DMA,
        ],
        compiler_params=params,
    )(_warp_body)
    out = f(table, flow.reshape(-1))
    return out.reshape(B, C, D, H, W)


def kernel(I, flow):
    return _warp(I, flow)


# speedup vs baseline: 10.1781x; 1.0848x over previous
import functools

import jax
import jax.numpy as jnp
from jax import lax
from jax.experimental import pallas as pl
from jax.experimental.pallas import tpu as pltpu
from jax.experimental.pallas import tpu_sc as plsc

B, C, D, H, W = 2, 1, 128, 192, 192
HW = H * W
DHW = D * HW
N = B * DHW

ZP, YP, XP = D + 1, H + 1, W + 1
RY = XP
RZ = YP * XP
ROWS_B = ZP * RZ

NC, NS, LANES = 2, 16, 16
NW = NC * NS
NPW = N // NW
WPB = NW // B
K = 2304
ROWS_PER_CHUNK = K // W
GROUPS_PER_ROW = W // LANES
CHUNKS = NPW // K

NR = 49
NY = NR + 1
NPLANES = B * ZP
NCH = (YP + NR - 1) // NR
NITEMS = NPLANES * NCH
TAIL = YP - (NCH - 1) * NR


def _build_body(I_hbm, table_hbm, src0, src1, outbuf):
    cid = lax.axis_index("c")
    sid = lax.axis_index("s")
    wid = sid * NC + cid

    iota16 = lax.iota(jnp.int32, LANES)

    def item_body(it, c1):
        w = wid + it * NW

        @pl.when(w < NITEMS)
        def _():
            plane = w // NCH
            ch = w - plane * NCH
            b = plane // ZP
            zp = plane - b * ZP
            yp0 = ch * NR
            full = ch < NCH - 1
            z0s = jnp.clip(zp - 1, 0, D - 1)
            z1s = jnp.clip(zp, 0, D - 1)
            ys0 = jnp.clip(yp0 - 1, 0, H - NY)
            ibase = b * DHW + ys0 * W
            pltpu.sync_copy(I_hbm.at[pl.ds(ibase + z0s * HW, NY * W)], src0)
            pltpu.sync_copy(I_hbm.at[pl.ds(ibase + z1s * HW, NY * W)], src1)

            def row_body(t):
                yp = yp0 + t
                b0 = (jnp.clip(yp - 1, 0, H - 1) - ys0) * W
                b1 = (jnp.clip(yp, 0, H - 1) - ys0) * W
                for g in range(13):
                    xpb = g * 16 if g < 12 else XP - 16
                    xv = xpb + iota16
                    xi0 = jnp.clip(xv - 1, 0, W - 1)
                    xi1 = jnp.clip(xv, 0, W - 1)
                    rows = t * XP + xv
                    combos = [(src, yb, xi)
                              for src in (src0, src1)
                              for yb in (b0, b1)
                              for xi in (xi0, xi1)]
                    vs = [plsc.load_gather(src, [yb + xi])
                          for (src, yb, xi) in combos]
                    for c, v in enumerate(vs):
                        plsc.store_scatter(
                            outbuf, [rows, jnp.full((LANES,), c, jnp.int32)], v)

            nrows = jnp.where(full, NR, TAIL)
            plsc.parallel_loop(0, nrows)(row_body)
            dst0 = ((b * ZP + zp) * YP + yp0) * XP

            @pl.when(full)
            def _():
                pltpu.sync_copy(outbuf.at[pl.ds(0, NR * XP), :],
                                table_hbm.at[pl.ds(dst0, NR * XP), :])

            @pl.when(jnp.logical_not(full))
            def _():
                pltpu.sync_copy(outbuf.at[pl.ds(0, TAIL * XP), :],
                                table_hbm.at[pl.ds(dst0, TAIL * XP), :])

        return c1

    lax.fori_loop(0, (NITEMS + NW - 1) // NW, item_body, 0)


def _floor_i32(v):
    ti = v.astype(jnp.int32)
    tf = ti.astype(jnp.float32)
    return jnp.where(tf > v, ti - 1, ti)


def _warp_body(table_hbm, flow_hbm, out_hbm, *rest):
    sets = [rest[i * 9:(i + 1) * 9] for i in range(3)]
    semf, semg, semo = rest[27:30]

    cid = lax.axis_index("c")
    sid = lax.axis_index("s")
    wid = sid * NC + cid
    b = wid // WPB
    widx = wid % WPB
    o_batch0 = widx * NPW
    obase = b * DHW
    tbase = b * ROWS_B
    fbase = b * 3 * DHW

    def flow_srcs(g):
        o = fbase + o_batch0 + g * K
        return (flow_hbm.at[pl.ds(o, K)],
                flow_hbm.at[pl.ds(o + DHW, K)],
                flow_hbm.at[pl.ds(o + 2 * DHW, K)])

    def fire_flow(g, S):
        s0, s1, s2 = flow_srcs(g)
        pltpu.async_copy(s0, S[0], semf)
        pltpu.async_copy(s1, S[1], semf)
        pltpu.async_copy(s2, S[2], semf)

    def wait_flow(g, S):
        s0, s1, s2 = flow_srcs(g)
        pltpu.make_async_copy(s0, S[0], semf).wait()
        pltpu.make_async_copy(s1, S[1], semf).wait()
        pltpu.make_async_copy(s2, S[2], semf).wait()

    def fire_gather(S):
        pltpu.async_copy(table_hbm.at[S[6]], S[7], semg)

    def drain_gather(S):
        pltpu.make_async_copy(table_hbm.at[S[6]], S[7], semg).wait()

    def out_dst(g):
        return out_hbm.at[pl.ds(obase + o_batch0 + g * K, K)]

    def fire_out(g, S):
        pltpu.async_copy(S[8], out_dst(g), semo)

    def wait_out(g, S):
        pltpu.make_async_copy(S[8], out_dst(g), semo).wait()

    def pass1(g, S):
        dxv, dyv, dzv, fxv, fyv, fzv, ixv = S[0], S[1], S[2], S[3], S[4], S[5], S[6]
        o = o_batch0 + g * K
        row0 = o // W

        def row_body(t):
            r = row0 + t
            z = r // H
            y = r - z * H
            yf = y.astype(jnp.float32)
            zf = z.astype(jnp.float32)
            for j in range(GROUPS_PER_ROW):
                sl = pl.ds(t * W + j * LANES, LANES)
                lane = lax.iota(jnp.int32, LANES).astype(jnp.float32) + float(j * LANES)
                xs = dxv[sl] + lane
                ys = dyv[sl] + yf
                zs = dzv[sl] + zf
                x0 = _floor_i32(xs)
                y0 = _floor_i32(ys)
                z0 = _floor_i32(zs)
                xp = jnp.clip(x0 + 1, 0, W)
                yp = jnp.clip(y0 + 1, 0, H)
                zp = jnp.clip(z0 + 1, 0, D)
                fxv[sl] = jnp.minimum(xp, W - 1).astype(jnp.float32) - xs
                fyv[sl] = jnp.minimum(yp, H - 1).astype(jnp.float32) - ys
                fzv[sl] = jnp.minimum(zp, D - 1).astype(jnp.float32) - zs
                ixv[sl] = zp * RZ + yp * RY + xp + tbase

        plsc.parallel_loop(0, ROWS_PER_CHUNK)(row_body)

    iota16 = lax.iota(jnp.int32, LANES)
    cols = [jnp.full((LANES,), c, jnp.int32) for c in range(8)]

    def combine(S):
        fxv, fyv, fzv, vlv, otv = S[3], S[4], S[5], S[7], S[8]

        def comb_body(i):
            sl = pl.ds(i * LANES, LANES)
            rows = i * LANES + iota16
            g = [plsc.load_gather(vlv, [rows, cols[c]]) for c in range(8)]
            fx = fxv[sl]
            fy = fyv[sl]
            fz = fzv[sl]
            ex = 1.0 - fx
            ey = 1.0 - fy
            ez = 1.0 - fz
            s00 = g[0] * fx + g[1] * ex
            s01 = g[2] * fx + g[3] * ex
            s10 = g[4] * fx + g[5] * ex
            s11 = g[6] * fx + g[7] * ex
            r0 = s00 * fy + s01 * ey
            r1 = s10 * fy + s11 * ey
            otv[sl] = r0 * fz + r1 * ez

        plsc.parallel_loop(0, K // LANES)(comb_body)

    def steady(g, cur, prv2, nxt):
        @pl.when(g < CHUNKS)
        def _():
            wait_flow(g, cur)
            pass1(g, cur)
            fire_gather(cur)

        @pl.when(g >= 2)
        def _():
            drain_gather(prv2)

            @pl.when(g >= 5)
            def _():
                wait_out(g - 5, prv2)

            combine(prv2)
            fire_out(g - 2, prv2)

        @pl.when(g + 1 < CHUNKS)
        def _():
            fire_flow(g + 1, nxt)

    S0, S1, S2 = sets
    SET = (S0, S1, S2)
    fire_flow(0, S0)

    def triple_body(p, c1):
        g = 3 * p
        steady(g, S0, S1, S1)
        steady(g + 1, S1, S2, S2)
        steady(g + 2, S2, S0, S0)
        return c1

    NTRIPLE = CHUNKS // 3
    lax.fori_loop(0, NTRIPLE, triple_body, 0)

    for g in range(3 * NTRIPLE, CHUNKS + 2):
        steady(jnp.int32(g), SET[g % 3], SET[(g - 2) % 3], SET[(g + 1) % 3])
    for g in range(CHUNKS - 3, CHUNKS):
        wait_out(g, SET[g % 3])


@jax.jit
def _warp(I, flow):
    mesh = plsc.VectorSubcoreMesh(core_axis_name="c", subcore_axis_name="s")
    params = pltpu.CompilerParams(
        needs_layout_passes=False, use_tc_tiling_on_sc=False)
    build = functools.partial(
        pl.kernel,
        mesh=mesh,
        out_type=jax.ShapeDtypeStruct((B * ROWS_B, 8), jnp.float32),
        scratch_types=[
            pltpu.VMEM((NY * W,), jnp.float32),
            pltpu.VMEM((NY * W,), jnp.float32),
            pltpu.VMEM((NR * XP, 8), jnp.float32),
        ],
        compiler_params=params,
    )(_build_body)
    table = build(I.reshape(-1))

    sbuf = [
        pltpu.VMEM((K,), jnp.float32),
        pltpu.VMEM((K,), jnp.float32),
        pltpu.VMEM((K,), jnp.float32),
        pltpu.VMEM((K,), jnp.float32),
        pltpu.VMEM((K,), jnp.float32),
        pltpu.VMEM((K,), jnp.float32),
        pltpu.VMEM((K,), jnp.int32),
        pltpu.VMEM((K, 8), jnp.float32),
        pltpu.VMEM((K,), jnp.float32),
    ]
    f = functools.partial(
        pl.kernel,
        mesh=mesh,
        out_type=jax.ShapeDtypeStruct((N,), jnp.float32),
        scratch_types=sbuf * 3 + [
            pltpu.SemaphoreType.DMA,
            pltpu.SemaphoreType.DMA,
            pltpu.SemaphoreType.DMA,
        ],
        compiler_params=params,
    )(_warp_body)
    out = f(table, flow.reshape(-1))
    return out.reshape(B, C, D, H, W)


def kernel(I, flow):
    return _warp(I, flow)


# speedup vs baseline: 11.3150x; 1.1117x over previous
import functools

import jax
import jax.numpy as jnp
from jax import lax
from jax.experimental import pallas as pl
from jax.experimental.pallas import tpu as pltpu
from jax.experimental.pallas import tpu_sc as plsc

B, C, D, H, W = 2, 1, 128, 192, 192
HW = H * W
DHW = D * HW
N = B * DHW

ZP, YP, XP = D + 1, H + 1, W + 1
RY = XP
RZ = YP * XP
ROWS_B = ZP * RZ

NC, NS, LANES = 2, 16, 16
NW = NC * NS
NPW = N // NW
WPB = NW // B
K = 2304
ROWS_PER_CHUNK = K // W
GROUPS_PER_ROW = W // LANES
CHUNKS = NPW // K

NR = 25
NY = NR + 1
NPLANES = B * ZP
NCH = (YP + NR - 1) // NR
NITEMS = NPLANES * NCH
TAIL = YP - (NCH - 1) * NR
NIT = (NITEMS + NW - 1) // NW


def _build_body(I_hbm, table_hbm, s0a, s1a, s0b, s1b, oba, obb, sems, semo):
    cid = lax.axis_index("c")
    sid = lax.axis_index("s")
    wid = sid * NC + cid

    iota16 = lax.iota(jnp.int32, LANES)
    srcs = ((s0a, s1a), (s0b, s1b))
    obufs = (oba, obb)

    def params(w):
        plane = w // NCH
        ch = w - plane * NCH
        b = plane // ZP
        zp = plane - b * ZP
        yp0 = ch * NR
        full = ch < NCH - 1
        ys0 = jnp.clip(yp0 - 1, 0, H - NY)
        ibase = b * DHW + ys0 * W
        s0 = I_hbm.at[pl.ds(ibase + jnp.clip(zp - 1, 0, D - 1) * HW, NY * W)]
        s1 = I_hbm.at[pl.ds(ibase + jnp.clip(zp, 0, D - 1) * HW, NY * W)]
        dst0 = ((b * ZP + zp) * YP + yp0) * XP
        return yp0, full, ys0, s0, s1, dst0

    def out_dsts(dst0):
        return (table_hbm.at[pl.ds(dst0, NR * XP), :],
                table_hbm.at[pl.ds(dst0, TAIL * XP), :])

    def fire_src(w, S):
        yp0, full, ys0, s0, s1, dst0 = params(w)
        pltpu.async_copy(s0, S[0], sems)
        pltpu.async_copy(s1, S[1], sems)

    def wait_src(w, S):
        yp0, full, ys0, s0, s1, dst0 = params(w)
        pltpu.make_async_copy(s0, S[0], sems).wait()
        pltpu.make_async_copy(s1, S[1], sems).wait()

    def fire_out(w, ob):
        yp0, full, ys0, s0, s1, dst0 = params(w)
        dfull, dtail = out_dsts(dst0)

        @pl.when(full)
        def _():
            pltpu.async_copy(ob.at[pl.ds(0, NR * XP), :], dfull, semo)

        @pl.when(jnp.logical_not(full))
        def _():
            pltpu.async_copy(ob.at[pl.ds(0, TAIL * XP), :], dtail, semo)

    def wait_out(w, ob):
        yp0, full, ys0, s0, s1, dst0 = params(w)
        dfull, dtail = out_dsts(dst0)

        @pl.when(full)
        def _():
            pltpu.make_async_copy(ob.at[pl.ds(0, NR * XP), :], dfull, semo).wait()

        @pl.when(jnp.logical_not(full))
        def _():
            pltpu.make_async_copy(ob.at[pl.ds(0, TAIL * XP), :], dtail, semo).wait()

    def interleave(w, S, ob):
        yp0, full, ys0, s0, s1, dst0 = params(w)
        src0, src1 = S

        def row_body(t):
            yp = yp0 + t
            b0 = (jnp.clip(yp - 1, 0, H - 1) - ys0) * W
            b1 = (jnp.clip(yp, 0, H - 1) - ys0) * W
            for g in range(13):
                xpb = g * 16 if g < 12 else XP - 16
                xv = xpb + iota16
                xi0 = jnp.clip(xv - 1, 0, W - 1)
                xi1 = jnp.clip(xv, 0, W - 1)
                rows = t * XP + xv
                combos = [(src, yb, xi)
                          for src in (src0, src1)
                          for yb in (b0, b1)
                          for xi in (xi0, xi1)]
                vs = [plsc.load_gather(src, [yb + xi])
                      for (src, yb, xi) in combos]
                for c, v in enumerate(vs):
                    plsc.store_scatter(
                        ob, [rows, jnp.full((LANES,), c, jnp.int32)], v)

        nrows = jnp.where(full, NR, TAIL)
        plsc.parallel_loop(0, nrows)(row_body)

    def cond_call(w, f, *args):
        @pl.when(jnp.logical_and(w >= 0, w < NITEMS))
        def _():
            f(w, *args)

    def pair_body(p, c1):
        for par in (0, 1):
            it = 2 * p + par
            w = wid + it * NW
            cond_call(w + NW, fire_src, srcs[1 - par])
            cond_call(w, wait_src, srcs[par])
            cond_call(w - 2 * NW, wait_out, obufs[par])
            cond_call(w, interleave, srcs[par], obufs[par])
            cond_call(w, fire_out, obufs[par])
        return c1

    cond_call(wid, fire_src, srcs[0])
    assert NIT % 2 == 1
    lax.fori_loop(0, (NIT + 1) // 2, pair_body, 0)
    for it in (NIT - 1, NIT):
        w = wid + it * NW
        cond_call(w, wait_out, obufs[it % 2])


def _floor_i32(v):
    ti = v.astype(jnp.int32)
    tf = ti.astype(jnp.float32)
    return jnp.where(tf > v, ti - 1, ti)


def _warp_body(table_hbm, flow_hbm, out_hbm, *rest):
    sets = [rest[i * 9:(i + 1) * 9] for i in range(3)]
    semf, semg, semo = rest[27:30]

    cid = lax.axis_index("c")
    sid = lax.axis_index("s")
    wid = sid * NC + cid
    b = wid // WPB
    widx = wid % WPB
    o_batch0 = widx * NPW
    obase = b * DHW
    tbase = b * ROWS_B
    fbase = b * 3 * DHW

    def flow_srcs(g):
        o = fbase + o_batch0 + g * K
        return (flow_hbm.at[pl.ds(o, K)],
                flow_hbm.at[pl.ds(o + DHW, K)],
                flow_hbm.at[pl.ds(o + 2 * DHW, K)])

    def fire_flow(g, S):
        s0, s1, s2 = flow_srcs(g)
        pltpu.async_copy(s0, S[0], semf)
        pltpu.async_copy(s1, S[1], semf)
        pltpu.async_copy(s2, S[2], semf)

    def wait_flow(g, S):
        s0, s1, s2 = flow_srcs(g)
        pltpu.make_async_copy(s0, S[0], semf).wait()
        pltpu.make_async_copy(s1, S[1], semf).wait()
        pltpu.make_async_copy(s2, S[2], semf).wait()

    def fire_gather(S):
        pltpu.async_copy(table_hbm.at[S[6]], S[7], semg)

    def drain_gather(S):
        pltpu.make_async_copy(table_hbm.at[S[6]], S[7], semg).wait()

    def out_dst(g):
        return out_hbm.at[pl.ds(obase + o_batch0 + g * K, K)]

    def fire_out(g, S):
        pltpu.async_copy(S[8], out_dst(g), semo)

    def wait_out(g, S):
        pltpu.make_async_copy(S[8], out_dst(g), semo).wait()

    def pass1(g, S):
        dxv, dyv, dzv, fxv, fyv, fzv, ixv = S[0], S[1], S[2], S[3], S[4], S[5], S[6]
        o = o_batch0 + g * K
        row0 = o // W

        def row_body(t):
            r = row0 + t
            z = r // H
            y = r - z * H
            yf = y.astype(jnp.float32)
            zf = z.astype(jnp.float32)
            for j in range(GROUPS_PER_ROW):
                sl = pl.ds(t * W + j * LANES, LANES)
                lane = lax.iota(jnp.int32, LANES).astype(jnp.float32) + float(j * LANES)
                xs = dxv[sl] + lane
                ys = dyv[sl] + yf
                zs = dzv[sl] + zf
                x0 = _floor_i32(xs)
                y0 = _floor_i32(ys)
                z0 = _floor_i32(zs)
                xp = jnp.clip(x0 + 1, 0, W)
                yp = jnp.clip(y0 + 1, 0, H)
                zp = jnp.clip(z0 + 1, 0, D)
                fxv[sl] = jnp.minimum(xp, W - 1).astype(jnp.float32) - xs
                fyv[sl] = jnp.minimum(yp, H - 1).astype(jnp.float32) - ys
                fzv[sl] = jnp.minimum(zp, D - 1).astype(jnp.float32) - zs
                ixv[sl] = zp * RZ + yp * RY + xp + tbase

        plsc.parallel_loop(0, ROWS_PER_CHUNK)(row_body)

    iota16 = lax.iota(jnp.int32, LANES)
    cols = [jnp.full((LANES,), c, jnp.int32) for c in range(8)]

    def combine(S):
        fxv, fyv, fzv, vlv, otv = S[3], S[4], S[5], S[7], S[8]

        def comb_body(i):
            sl = pl.ds(i * LANES, LANES)
            rows = i * LANES + iota16
            g = [plsc.load_gather(vlv, [rows, cols[c]]) for c in range(8)]
            fx = fxv[sl]
            fy = fyv[sl]
            fz = fzv[sl]
            ex = 1.0 - fx
            ey = 1.0 - fy
            ez = 1.0 - fz
            s00 = g[0] * fx + g[1] * ex
            s01 = g[2] * fx + g[3] * ex
            s10 = g[4] * fx + g[5] * ex
            s11 = g[6] * fx + g[7] * ex
            r0 = s00 * fy + s01 * ey
            r1 = s10 * fy + s11 * ey
            otv[sl] = r0 * fz + r1 * ez

        plsc.parallel_loop(0, K // LANES)(comb_body)

    def steady(g, cur, prv2, nxt):
        @pl.when(g < CHUNKS)
        def _():
            wait_flow(g, cur)
            pass1(g, cur)
            fire_gather(cur)

        @pl.when(g >= 2)
        def _():
            drain_gather(prv2)

            @pl.when(g >= 5)
            def _():
                wait_out(g - 5, prv2)

            combine(prv2)
            fire_out(g - 2, prv2)

        @pl.when(g + 1 < CHUNKS)
        def _():
            fire_flow(g + 1, nxt)

    S0, S1, S2 = sets
    SET = (S0, S1, S2)
    fire_flow(0, S0)

    def triple_body(p, c1):
        g = 3 * p
        steady(g, S0, S1, S1)
        steady(g + 1, S1, S2, S2)
        steady(g + 2, S2, S0, S0)
        return c1

    NTRIPLE = CHUNKS // 3
    lax.fori_loop(0, NTRIPLE, triple_body, 0)

    for g in range(3 * NTRIPLE, CHUNKS + 2):
        steady(jnp.int32(g), SET[g % 3], SET[(g - 2) % 3], SET[(g + 1) % 3])
    for g in range(CHUNKS - 3, CHUNKS):
        wait_out(g, SET[g % 3])


@jax.jit
def _warp(I, flow):
    mesh = plsc.VectorSubcoreMesh(core_axis_name="c", subcore_axis_name="s")
    params = pltpu.CompilerParams(
        needs_layout_passes=False, use_tc_tiling_on_sc=False)
    build = functools.partial(
        pl.kernel,
        mesh=mesh,
        out_type=jax.ShapeDtypeStruct((B * ROWS_B, 8), jnp.float32),
        scratch_types=[
            pltpu.VMEM((NY * W,), jnp.float32),
            pltpu.VMEM((NY * W,), jnp.float32),
            pltpu.VMEM((NY * W,), jnp.float32),
            pltpu.VMEM((NY * W,), jnp.float32),
            pltpu.VMEM((NR * XP, 8), jnp.float32),
            pltpu.VMEM((NR * XP, 8), jnp.float32),
            pltpu.SemaphoreType.DMA,
            pltpu.SemaphoreType.DMA,
        ],
        compiler_params=params,
    )(_build_body)
    table = build(I.reshape(-1))

    sbuf = [
        pltpu.VMEM((K,), jnp.float32),
        pltpu.VMEM((K,), jnp.float32),
        pltpu.VMEM((K,), jnp.float32),
        pltpu.VMEM((K,), jnp.float32),
        pltpu.VMEM((K,), jnp.float32),
        pltpu.VMEM((K,), jnp.float32),
        pltpu.VMEM((K,), jnp.int32),
        pltpu.VMEM((K, 8), jnp.float32),
        pltpu.VMEM((K,), jnp.float32),
    ]
    f = functools.partial(
        pl.kernel,
        mesh=mesh,
        out_type=jax.ShapeDtypeStruct((N,), jnp.float32),
        scratch_types=sbuf * 3 + [
            pltpu.SemaphoreType.DMA,
            pltpu.SemaphoreType.DMA,
            pltpu.SemaphoreType.DMA,
        ],
        compiler_params=params,
    )(_warp_body)
    out = f(table, flow.reshape(-1))
    return out.reshape(B, C, D, H, W)


def kernel(I, flow):
    return _warp(I, flow)
